# baseline (speedup 1.0000x reference)
import functools
import math

import jax
import jax.numpy as jnp
from jax import lax
from jax.experimental import pallas as pl
from jax.experimental.pallas import tpu as pltpu
from jax.experimental.pallas import tpu_sc as plsc

VOCAB = 100000
DIM = 64
SEQ = 1500
BATCH = 1024
SEQ_PAD = 1504

NC = 2
NS = 16
NW = NC * NS
ROWS_PER_W = BATCH // NW

CHUNKS = ((0, 512), (512, 512), (1024, 476))
CLMAX = 512
IGS = 128


def _pe_table():
    position = jnp.arange(SEQ, dtype=jnp.float32)[:, None]
    div_term = jnp.exp(
        jnp.arange(0, DIM, 2, dtype=jnp.float32) * (-(math.log(10000.0) / DIM))
    )
    ang = position * div_term
    pe = jnp.zeros((SEQ, DIM), dtype=jnp.float32)
    pe = pe.at[:, 0::2].set(jnp.sin(ang))
    pe = pe.at[:, 1::2].set(jnp.cos(ang))
    return pe


@functools.partial(
    pl.kernel,
    mesh=plsc.VectorSubcoreMesh(core_axis_name="c", subcore_axis_name="s"),
    out_type=jax.ShapeDtypeStruct((BATCH, SEQ, DIM), jnp.float32),
    scratch_types=(
        [pltpu.VMEM((CLMAX, DIM), jnp.float32)]
        + [pltpu.VMEM((CLMAX,), jnp.int32) for _ in range(4)]
        + [pltpu.VMEM((CLMAX, DIM), jnp.float32) for _ in range(2)]
        + [pltpu.SemaphoreType.DMA for _ in range(4)]
        + [pltpu.SemaphoreType.DMA for _ in range(8)]
        + [pltpu.SemaphoreType.DMA for _ in range(2)]
    ),
    compiler_params=pltpu.CompilerParams(use_tc_tiling_on_sc=False),
)
def _seq_encode(tok_hbm, pe_hbm, table_hbm, out_hbm, pe_v, *scratch):
    idx = scratch[0:4]
    rows = scratch[4:6]
    sem_i = scratch[6:10]
    sem_g = (scratch[10:14], scratch[14:18])
    sem_s = scratch[18:20]
    wid = lax.axis_index("s") * NC + lax.axis_index("c")

    for off, cl in CHUNKS:
        cl_pad = -(-cl // 8) * 8
        n_g, rem = divmod(cl_pad, IGS)

        pltpu.sync_copy(pe_hbm.at[pl.ds(off, cl), :], pe_v.at[pl.ds(0, cl), :])

        def fire_idx(g, b, off=off, cl_pad=cl_pad):
            pltpu.async_copy(
                tok_hbm.at[pl.ds(g * SEQ_PAD + off, cl_pad)],
                idx[b].at[pl.ds(0, cl_pad)], sem_i[b])

        def drain_idx(b, cl_pad=cl_pad):
            pltpu.make_async_copy(
                tok_hbm.at[pl.ds(0, cl_pad)],
                idx[b].at[pl.ds(0, cl_pad)], sem_i[b]).wait()

        def fire_gathers(b, rb, n_g=n_g, rem=rem):
            for j in range(n_g):
                pltpu.async_copy(
                    table_hbm.at[idx[b].at[pl.ds(j * IGS, IGS)]],
                    rows[rb].at[pl.ds(j * IGS, IGS), :], sem_g[rb][j])
            if rem:
                pltpu.async_copy(
                    table_hbm.at[idx[b].at[pl.ds(n_g * IGS, rem)]],
                    rows[rb].at[pl.ds(n_g * IGS, rem), :], sem_g[rb][n_g])

        def drain_gather(rb, j, n_g=n_g, rem=rem):
            n = IGS if j < n_g else rem
            pltpu.make_async_copy(
                pe_hbm.at[pl.ds(0, n), :],
                rows[rb].at[pl.ds(j * IGS, n), :], sem_g[rb][j]).wait()

        def add_block(rb, base, n):
            unroll = 8 if n % 8 == 0 else 4

            def add_body(i, c):
                for u in range(unroll):
                    p = base + i * unroll + u
                    for v in range(DIM // 16):
                        plsc.addupdate(
                            rows[rb].at[p, pl.ds(v * 16, 16)],
                            pe_v[p, pl.ds(v * 16, 16)],
                        )
                return c
            lax.fori_loop(0, n // unroll, add_body, 0)

        blocks = []
        done = 0
        for j in range(n_g + (1 if rem else 0)):
            n = min(IGS, cl - done)
            blocks.append((j, done, n))
            done += n

        def drain_add(rb):
            for j, base, n in blocks:
                drain_gather(rb, j)
                add_block(rb, base, n)

        def fire_store(g, rb, off=off, cl=cl):
            pltpu.async_copy(
                rows[rb].at[pl.ds(0, cl), :],
                out_hbm.at[g, pl.ds(off, cl), :], sem_s[rb])

        def drain_store(rb, off=off, cl=cl):
            pltpu.make_async_copy(
                rows[rb].at[pl.ds(0, cl), :],
                out_hbm.at[0, pl.ds(off, cl), :], sem_s[rb]).wait()

        def do_pair(p2, ia, ib):
            ga = wid * ROWS_PER_W + 2 * p2
            gb = ga + 1

            @pl.when(p2 > 0)
            def _():
                drain_store(0)
            drain_idx(ia)
            fire_gathers(ia, 0)

            @pl.when(p2 > 0)
            def _():
                drain_store(1)
            drain_idx(ib)
            fire_gathers(ib, 1)

            @pl.when(2 * p2 + 2 < ROWS_PER_W)
            def _():
                fire_idx(ga + 2, (ia + 2) % 4)
                fire_idx(gb + 2, (ib + 2) % 4)

            drain_add(0)
            fire_store(ga, 0)

            drain_add(1)
            fire_store(gb, 1)

        fire_idx(wid * ROWS_PER_W, 0)
        fire_idx(wid * ROWS_PER_W + 1, 1)

        def quad_body(i, carry):
            do_pair(2 * i, 0, 1)
            do_pair(2 * i + 1, 2, 3)
            return carry

        lax.fori_loop(0, ROWS_PER_W // 4, quad_body, 0)
        drain_store(0)
        drain_store(1)


def kernel(tokens, table):
    pe = _pe_table()
    tok_flat = jnp.pad(tokens, ((0, 0), (0, SEQ_PAD - SEQ))).reshape(-1)
    return _seq_encode(tok_flat, pe, table)


# speedup vs baseline: 1.0138x; 1.0138x over previous
import functools
import math

import jax
import jax.numpy as jnp
from jax import lax
from jax.experimental import pallas as pl
from jax.experimental.pallas import tpu as pltpu
from jax.experimental.pallas import tpu_sc as plsc

VOCAB = 100000
DIM = 64
SEQ = 1500
BATCH = 1024
SEQ_PAD = 1504

NC = 2
NS = 16
NW = NC * NS
ROWS_PER_W = BATCH // NW

CHUNKS = ((0, 320), (320, 320), (640, 320), (960, 320), (1280, 220))
CLMAX = 320
IGS = 128
NSTREAM = 3


def _pe_table():
    position = jnp.arange(SEQ, dtype=jnp.float32)[:, None]
    div_term = jnp.exp(
        jnp.arange(0, DIM, 2, dtype=jnp.float32) * (-(math.log(10000.0) / DIM))
    )
    ang = position * div_term
    pe = jnp.zeros((SEQ, DIM), dtype=jnp.float32)
    pe = pe.at[:, 0::2].set(jnp.sin(ang))
    pe = pe.at[:, 1::2].set(jnp.cos(ang))
    return pe


@functools.partial(
    pl.kernel,
    mesh=plsc.VectorSubcoreMesh(core_axis_name="c", subcore_axis_name="s"),
    out_type=jax.ShapeDtypeStruct((BATCH, SEQ, DIM), jnp.float32),
    scratch_types=(
        [pltpu.VMEM((CLMAX, DIM), jnp.float32)]
        + [pltpu.VMEM((CLMAX,), jnp.int32) for _ in range(4)]
        + [pltpu.VMEM((CLMAX, DIM), jnp.float32) for _ in range(4)]
        + [pltpu.SemaphoreType.DMA for _ in range(4)]
        + [pltpu.SemaphoreType.DMA for _ in range(4 * NSTREAM)]
        + [pltpu.SemaphoreType.DMA for _ in range(4)]
    ),
    compiler_params=pltpu.CompilerParams(use_tc_tiling_on_sc=False),
)
def _seq_encode(tok_hbm, pe_hbm, table_hbm, out_hbm, pe_v, *scratch):
    idx = scratch[0:4]
    rows = scratch[4:8]
    sem_i = scratch[8:12]
    sem_g = [scratch[12 + NSTREAM * b: 12 + NSTREAM * (b + 1)] for b in range(4)]
    sem_s = scratch[12 + 4 * NSTREAM: 16 + 4 * NSTREAM]
    wid = lax.axis_index("s") * NC + lax.axis_index("c")

    for off, cl in CHUNKS:
        cl_pad = -(-cl // 8) * 8
        n_g, rem = divmod(cl_pad, IGS)
        nst = n_g + (1 if rem else 0)

        blocks = []
        done = 0
        for j in range(nst):
            gn = IGS if j < n_g else rem
            an = min(gn, cl - done)
            blocks.append((j, done, gn, an))
            done += gn

        pltpu.sync_copy(pe_hbm.at[pl.ds(off, cl), :], pe_v.at[pl.ds(0, cl), :])

        def fire_idx(g, b, off=off, cl_pad=cl_pad):
            pltpu.async_copy(
                tok_hbm.at[pl.ds(g * SEQ_PAD + off, cl_pad)],
                idx[b].at[pl.ds(0, cl_pad)], sem_i[b])

        def drain_idx(b, cl_pad=cl_pad):
            pltpu.make_async_copy(
                tok_hbm.at[pl.ds(0, cl_pad)],
                idx[b].at[pl.ds(0, cl_pad)], sem_i[b]).wait()

        def fire_gathers(b, blocks=blocks):
            for j, base, gn, an in blocks:
                pltpu.async_copy(
                    table_hbm.at[idx[b].at[pl.ds(base, gn)]],
                    rows[b].at[pl.ds(base, gn), :], sem_g[b][j])

        def add_block(b, base, n):
            unroll = 8 if n % 8 == 0 else 4

            def add_body(i, c):
                for u in range(unroll):
                    p = base + i * unroll + u
                    for v in range(DIM // 16):
                        plsc.addupdate(
                            rows[b].at[p, pl.ds(v * 16, 16)],
                            pe_v[p, pl.ds(v * 16, 16)],
                        )
                return c
            lax.fori_loop(0, n // unroll, add_body, 0)

        def drain_add(b, blocks=blocks):
            for j, base, gn, an in blocks:
                pltpu.make_async_copy(
                    pe_hbm.at[pl.ds(0, gn), :],
                    rows[b].at[pl.ds(base, gn), :], sem_g[b][j]).wait()
                add_block(b, base, an)

        def fire_store(g, b, off=off, cl=cl):
            pltpu.async_copy(
                rows[b].at[pl.ds(0, cl), :],
                out_hbm.at[g, pl.ds(off, cl), :], sem_s[b])

        def drain_store(b, off=off, cl=cl):
            pltpu.make_async_copy(
                rows[b].at[pl.ds(0, cl), :],
                out_hbm.at[0, pl.ds(off, cl), :], sem_s[b]).wait()

        g0 = wid * ROWS_PER_W
        for b in range(4):
            fire_idx(g0 + b, b)
        drain_idx(0)
        fire_gathers(0)
        drain_idx(1)
        fire_gathers(1)

        def quad_body(i, carry):
            r4 = 4 * i
            ga = g0 + r4

            @pl.when(r4 > 0)
            def _():
                drain_store(2)
                drain_store(3)
            drain_idx(2)
            fire_gathers(2)
            drain_idx(3)
            fire_gathers(3)

            drain_add(0)
            fire_store(ga, 0)
            drain_add(1)
            fire_store(ga + 1, 1)

            @pl.when(r4 + 4 < ROWS_PER_W)
            def _():
                fire_idx(ga + 4, 0)
                fire_idx(ga + 5, 1)

            drain_add(2)
            fire_store(ga + 2, 2)
            drain_add(3)
            fire_store(ga + 3, 3)

            @pl.when(r4 + 4 < ROWS_PER_W)
            def _():
                fire_idx(ga + 6, 2)
                fire_idx(ga + 7, 3)

            drain_store(0)
            drain_store(1)

            @pl.when(r4 + 4 < ROWS_PER_W)
            def _():
                drain_idx(0)
                fire_gathers(0)
                drain_idx(1)
                fire_gathers(1)
            return carry

        lax.fori_loop(0, ROWS_PER_W // 4, quad_body, 0)
        drain_store(2)
        drain_store(3)


def kernel(tokens, table):
    pe = _pe_table()
    tok_flat = jnp.pad(tokens, ((0, 0), (0, SEQ_PAD - SEQ))).reshape(-1)
    return _seq_encode(tok_flat, pe, table)
